# baseline (speedup 1.0000x reference)
import jax
import jax.numpy as jnp
from jax import lax
from jax.experimental import pallas as pl
from jax.experimental.pallas import tpu as pltpu
from jax.experimental.pallas import tpu_sc as plsc

B, C, H, W = 8, 192, 224, 224
HW = H * W
G = 8
RL = HW // G
NROWS = B * C * G
NW = 32
CPW = (B * C) // NW
WPB = C // CPW
CH = 4
NBUF = 4
NCK = CPW * G // CH
NWAVES = NCK // NBUF


def _shuffle_body(x_hbm, perm_hbm, out_hbm, pbuf, idx_v, bufs, gsems, ssems):
    cid = lax.axis_index("c")
    sid = lax.axis_index("s")
    wid = sid * 2 + cid
    b = wid // WPB
    cbase = (wid % WPB) * CPW

    pltpu.sync_copy(perm_hbm.at[pl.ds(pl.multiple_of(cbase, 8), CPW)], pbuf)

    lanes = lax.iota(jnp.int32, 16)
    sub = ((lanes >> 3) << 2) + (lanes & 3)
    for gblk in range(CPW // 16):
        pv = pbuf[pl.ds(16 * gblk, 16)]
        for l in range(16):
            m = 16 * gblk + l
            srow = (b * C + pv[l]) * G
            idx_v[pl.ds(16 * m, 16)] = srow + sub

    base = wid * CPW * G

    def fire_gather(i, buf, sem):
        src = x_hbm.at[idx_v.at[pl.ds(pl.multiple_of(8 * i, 8), CH)]]
        return pltpu.async_copy(src, buf, sem)

    def fire_scatter(i, buf, sem):
        dst = out_hbm.at[pl.ds(pl.multiple_of(base + CH * i, CH), CH)]
        pltpu.async_copy(buf, dst, sem)

    def wait_scatter(sem):
        pltpu.make_async_copy(bufs[0], out_hbm.at[pl.ds(0, CH)], sem).wait()

    gds = [fire_gather(t, bufs[t], gsems[t]) for t in range(NBUF)]
    for t in range(NBUF):
        gds[t].wait()
        fire_scatter(t, bufs[t], ssems[t])

    def body(k, carry):
        i0 = k * NBUF
        gds = []
        for t in range(NBUF):
            wait_scatter(ssems[t])
            gds.append(fire_gather(i0 + t, bufs[t], gsems[t]))
        for t in range(NBUF):
            gds[t].wait()
            fire_scatter(i0 + t, bufs[t], ssems[t])
        return carry

    lax.fori_loop(1, NWAVES, body, 0)
    for t in range(NBUF):
        wait_scatter(ssems[t])


@jax.jit
def _shuffle(x2, perm):
    mesh = plsc.VectorSubcoreMesh(core_axis_name="c", subcore_axis_name="s")
    return pl.kernel(
        _shuffle_body,
        out_type=jax.ShapeDtypeStruct((NROWS, RL), jnp.float32),
        mesh=mesh,
        scratch_types=[
            pltpu.VMEM((CPW,), jnp.int32),
            pltpu.VMEM((8 * NCK,), jnp.int32),
            [pltpu.VMEM((CH, RL), jnp.float32) for _ in range(NBUF)],
            [pltpu.SemaphoreType.DMA for _ in range(NBUF)],
            [pltpu.SemaphoreType.DMA for _ in range(NBUF)],
        ],
    )(x2, perm)


def kernel(x, perm):
    x2 = x.reshape(NROWS, RL)
    y2 = _shuffle(x2, perm.astype(jnp.int32))
    return (y2.reshape(B, C, H, W), jnp.zeros((), dtype=jnp.float32))


# speedup vs baseline: 3.6363x; 3.6363x over previous
import jax
import jax.numpy as jnp
from jax import lax
from jax.experimental import pallas as pl
from jax.experimental.pallas import tpu as pltpu
from jax.experimental.pallas import tpu_sc as plsc

B, C, H, W = 8, 192, 224, 224
NW = 32
CPW = (B * C) // NW
WPB = C // CPW


def _shuffle_body(x_hbm, perm_hbm, out_hbm, pbuf, buf0, buf1,
                  g0, g1, s0, s1):
    cid = lax.axis_index("c")
    sid = lax.axis_index("s")
    wid = sid * 2 + cid
    b = wid // WPB
    cbase = (wid % WPB) * CPW

    pltpu.sync_copy(perm_hbm, pbuf)
    pvs = [pbuf[pl.ds(pl.multiple_of(cbase + 16 * t, 16), 16)]
           for t in range(CPW // 16)]

    def src_c(l):
        return pvs[l // 16][l % 16]

    bufs = (buf0, buf1)
    gsems = (g0, g1)
    ssems = (s0, s1)

    def fire_gather(l):
        return pltpu.async_copy(x_hbm.at[b, src_c(l)], bufs[l % 2],
                                gsems[l % 2])

    def fire_scatter(l):
        pltpu.async_copy(bufs[l % 2], out_hbm.at[b, cbase + l],
                         ssems[l % 2])

    def wait_scatter(sem):
        pltpu.make_async_copy(buf0, out_hbm.at[0, 0], sem).wait()

    gd = fire_gather(0)
    for l in range(CPW):
        if l + 1 < CPW:
            if l + 1 >= 2:
                wait_scatter(ssems[(l + 1) % 2])
            gd_next = fire_gather(l + 1)
        gd.wait()
        fire_scatter(l)
        if l + 1 < CPW:
            gd = gd_next
    wait_scatter(s0)
    wait_scatter(s1)


@jax.jit
def _shuffle(x, perm):
    mesh = plsc.VectorSubcoreMesh(core_axis_name="c", subcore_axis_name="s")
    return pl.kernel(
        _shuffle_body,
        out_type=jax.ShapeDtypeStruct((B, C, H, W), jnp.float32),
        mesh=mesh,
        compiler_params=pltpu.CompilerParams(use_tc_tiling_on_sc=True),
        scratch_types=[
            pltpu.VMEM((C,), jnp.int32),
            pltpu.VMEM((H, W), jnp.float32),
            pltpu.VMEM((H, W), jnp.float32),
            pltpu.SemaphoreType.DMA,
            pltpu.SemaphoreType.DMA,
            pltpu.SemaphoreType.DMA,
            pltpu.SemaphoreType.DMA,
        ],
    )(x, perm)


def kernel(x, perm):
    y = _shuffle(x, perm.astype(jnp.int32))
    return (y, jnp.zeros((), dtype=jnp.float32))


# speedup vs baseline: 3.6594x; 1.0063x over previous
import jax
import jax.numpy as jnp
from jax import lax
from jax.experimental import pallas as pl
from jax.experimental.pallas import tpu as pltpu
from jax.experimental.pallas import tpu_sc as plsc

B, C, H, W = 8, 192, 224, 224
NW = 32
CPW = (B * C) // NW
WPB = C // CPW
HALF = H // 2
NCK = 2 * CPW
NBUF = 4
LOOK = 2


def _shuffle_body(x_hbm, perm_hbm, out_hbm, pbuf, bufs, gsems, ssems):
    cid = lax.axis_index("c")
    sid = lax.axis_index("s")
    wid = sid * 2 + cid
    b = wid // WPB
    cbase = (wid % WPB) * CPW

    pltpu.sync_copy(perm_hbm, pbuf)
    pvs = [pbuf[pl.ds(pl.multiple_of(cbase + 16 * t, 16), 16)]
           for t in range(CPW // 16)]

    def src_c(l):
        return pvs[l // 16][l % 16]

    def fire_gather(i):
        l, h = i // 2, i % 2
        src = x_hbm.at[b, src_c(l), pl.ds(HALF * h, HALF)]
        return pltpu.async_copy(src, bufs[i % NBUF], gsems[i % NBUF])

    def fire_scatter(i):
        l, h = i // 2, i % 2
        dst = out_hbm.at[b, cbase + l, pl.ds(HALF * h, HALF)]
        pltpu.async_copy(bufs[i % NBUF], dst, ssems[i % NBUF])

    def wait_scatter(sem):
        pltpu.make_async_copy(bufs[0], out_hbm.at[0, 0, pl.ds(0, HALF)],
                              sem).wait()

    gds = {i: fire_gather(i) for i in range(LOOK)}
    for i in range(NCK):
        j = i + LOOK
        if j < NCK:
            if j >= NBUF:
                wait_scatter(ssems[j % NBUF])
            gds[j] = fire_gather(j)
        gds.pop(i).wait()
        fire_scatter(i)
    for t in range(NBUF):
        wait_scatter(ssems[t])


@jax.jit
def _shuffle(x, perm):
    mesh = plsc.VectorSubcoreMesh(core_axis_name="c", subcore_axis_name="s")
    return pl.kernel(
        _shuffle_body,
        out_type=jax.ShapeDtypeStruct((B, C, H, W), jnp.float32),
        mesh=mesh,
        compiler_params=pltpu.CompilerParams(use_tc_tiling_on_sc=True),
        scratch_types=[
            pltpu.VMEM((C,), jnp.int32),
            [pltpu.VMEM((HALF, W), jnp.float32) for _ in range(NBUF)],
            [pltpu.SemaphoreType.DMA for _ in range(NBUF)],
            [pltpu.SemaphoreType.DMA for _ in range(NBUF)],
        ],
    )(x, perm)


def kernel(x, perm):
    y = _shuffle(x, perm.astype(jnp.int32))
    return (y, jnp.zeros((), dtype=jnp.float32))


# speedup vs baseline: 3.8670x; 1.0567x over previous
import jax
import jax.numpy as jnp
from jax import lax
from jax.experimental import pallas as pl
from jax.experimental.pallas import tpu as pltpu
from jax.experimental.pallas import tpu_sc as plsc

B, C, H, W = 8, 192, 224, 224
NW = 32
CPW = (B * C) // NW
WPB = C // CPW
HALF = H // 2
NCK = CPW


def _shuffle_body(x_hbm, perm_hbm, out_hbm, pbuf, bufs, shared,
                  gsems, ssems, spg, sps):
    cid = lax.axis_index("c")
    sid = lax.axis_index("s")
    wid = sid * 2 + cid
    b = wid // WPB
    cbase = (wid % WPB) * CPW

    pltpu.sync_copy(perm_hbm, pbuf)
    pvs = [pbuf[pl.ds(pl.multiple_of(cbase + 16 * t, 16), 16)]
           for t in range(CPW // 16)]

    def src_c(l):
        return pvs[l // 16][l % 16]

    def chunk_refs(path, k):
        l, h = 2 * (k // 2) + path, k % 2
        src = x_hbm.at[b, src_c(l), pl.ds(HALF * h, HALF)]
        dst = out_hbm.at[b, cbase + l, pl.ds(HALF * h, HALF)]
        return src, dst

    def make_ring(path, buf_at, gsem, ssem):
        def fire_gather(k):
            src, _ = chunk_refs(path, k)
            return pltpu.async_copy(src, buf_at(k % 2), gsem[k % 2])

        def fire_scatter(k):
            _, dst = chunk_refs(path, k)
            pltpu.async_copy(buf_at(k % 2), dst, ssem[k % 2])

        def wait_scatter(sem):
            pltpu.make_async_copy(buf_at(0),
                                  out_hbm.at[0, 0, pl.ds(0, HALF)],
                                  sem).wait()
        return fire_gather, fire_scatter, wait_scatter

    ts = make_ring(0, lambda t: bufs[t], gsems, ssems)
    sp = make_ring(1, lambda t: shared.at[sid, t], spg, sps)

    gds = {("ts", 0): ts[0](0), ("sp", 0): sp[0](0)}
    for k in range(NCK):
        j = k + 1
        for name, (fg, fs, ws), sem in (("ts", ts, ssems), ("sp", sp, sps)):
            if j < NCK:
                if j >= 2:
                    ws(sem[j % 2])
                gds[(name, j)] = fg(j)
            gds.pop((name, k)).wait()
            fs(k)
    for name, (fg, fs, ws), sem in (("ts", ts, ssems), ("sp", sp, sps)):
        ws(sem[0])
        ws(sem[1])


@jax.jit
def _shuffle(x, perm):
    mesh = plsc.VectorSubcoreMesh(core_axis_name="c", subcore_axis_name="s")
    return pl.kernel(
        _shuffle_body,
        out_type=jax.ShapeDtypeStruct((B, C, H, W), jnp.float32),
        mesh=mesh,
        compiler_params=pltpu.CompilerParams(use_tc_tiling_on_sc=True),
        scratch_types=[
            pltpu.VMEM((C,), jnp.int32),
            [pltpu.VMEM((HALF, W), jnp.float32) for _ in range(2)],
            pltpu.VMEM_SHARED((16, 2, HALF, W), jnp.float32),
            [pltpu.SemaphoreType.DMA for _ in range(2)],
            [pltpu.SemaphoreType.DMA for _ in range(2)],
            [pltpu.SemaphoreType.DMA for _ in range(2)],
            [pltpu.SemaphoreType.DMA for _ in range(2)],
        ],
    )(x, perm)


def kernel(x, perm):
    y = _shuffle(x, perm.astype(jnp.int32))
    return (y, jnp.zeros((), dtype=jnp.float32))
